# baseline (speedup 1.0000x reference)
import functools

import jax
import jax.numpy as jnp
from jax import lax
from jax.experimental import pallas as pl
from jax.experimental.pallas import tpu as pltpu
from jax.experimental.pallas import tpu_sc as plsc

_EPS = 1e-12
_GATHER_WINDOW = 160


def _sc_gather(word_emb, flat_ids):
    n_tok = flat_ids.shape[0]
    emb = word_emb.shape[1]
    nc, ns = 2, 16
    nw = nc * ns
    n_per_w = n_tok // nw
    chunk = _GATHER_WINDOW
    n_chunks = n_per_w // chunk
    mesh = plsc.VectorSubcoreMesh(core_axis_name="c", subcore_axis_name="s")

    @functools.partial(
        pl.kernel,
        out_type=jax.ShapeDtypeStruct((n_tok, emb), jnp.float32),
        mesh=mesh,
        scratch_types=[
            pltpu.VMEM((n_per_w,), jnp.int32),
            pltpu.VMEM((chunk, emb), jnp.float32),
            pltpu.SemaphoreType.DMA,
        ],
    )
    def gather_kernel(tab_hbm, idx_hbm, o_hbm, idx_v, rows_v, sem):
        wid = lax.axis_index("s") * nc + lax.axis_index("c")
        base = wid * n_per_w
        pltpu.sync_copy(idx_hbm.at[pl.ds(base, n_per_w)], idx_v)

        @pl.loop(0, n_chunks)
        def _(j):
            off = j * chunk
            pltpu.async_copy(
                tab_hbm.at[idx_v.at[pl.ds(off, chunk)]], rows_v, sem
            ).wait()
            pltpu.sync_copy(rows_v, o_hbm.at[pl.ds(base + off, chunk)])

    return gather_kernel(word_emb, flat_ids)


def _tc_add_ln(gathered, tti_f, pe0, dt, gamma, beta, nb=8):
    b, s = tti_f.shape[:2]
    e = gathered.shape[-1]
    g3 = gathered.reshape(b, s, e)

    def body(g_ref, tti_ref, pe_ref, dt_ref, gam_ref, bet_ref, o_ref):
        x = g_ref[...] + pe_ref[...] + tti_ref[...] * dt_ref[...]
        mu = jnp.mean(x, axis=-1, keepdims=True)
        xc = x - mu
        var = jnp.mean(xc * xc, axis=-1, keepdims=True)
        y = xc * lax.rsqrt(var + _EPS)
        o_ref[...] = y * gam_ref[...] + bet_ref[...]

    return pl.pallas_call(
        body,
        grid=(b // nb,),
        in_specs=[
            pl.BlockSpec((nb, s, e), lambda i: (i, 0, 0)),
            pl.BlockSpec((nb, s, 1), lambda i: (i, 0, 0)),
            pl.BlockSpec((s, e), lambda i: (0, 0)),
            pl.BlockSpec((e,), lambda i: (0,)),
            pl.BlockSpec((e,), lambda i: (0,)),
            pl.BlockSpec((e,), lambda i: (0,)),
        ],
        out_specs=pl.BlockSpec((nb, s, e), lambda i: (i, 0, 0)),
        out_shape=jax.ShapeDtypeStruct((b, s, e), jnp.float32),
    )(g3, tti_f, pe0, dt, gamma, beta)


def kernel(token_ids, token_type_ids, word_emb, pos_emb, type_emb, ln_gamma, ln_beta):
    b, s = token_ids.shape
    e = word_emb.shape[1]
    flat_ids = token_ids.reshape(b * s).astype(jnp.int32)
    gathered = _sc_gather(word_emb, flat_ids)
    pe0 = lax.slice(pos_emb, (0, 0), (s, e)) + type_emb[0]
    dt = type_emb[1] - type_emb[0]
    tti_f = token_type_ids[..., None].astype(jnp.float32)
    return _tc_add_ln(gathered, tti_f, pe0, dt, ln_gamma, ln_beta)


# speedup vs baseline: 1.0994x; 1.0994x over previous
import functools

import jax
import jax.numpy as jnp
from jax import lax
from jax.experimental import pallas as pl
from jax.experimental.pallas import tpu as pltpu
from jax.experimental.pallas import tpu_sc as plsc

_EPS = 1e-12
_GATHER_WINDOW = 160


def _sc_gather(word_emb, flat_ids):
    n_tok = flat_ids.shape[0]
    emb = word_emb.shape[1]
    nc, ns = 2, 16
    nw = nc * ns
    n_per_w = n_tok // nw
    chunk = _GATHER_WINDOW
    n_chunks = n_per_w // chunk
    mesh = plsc.VectorSubcoreMesh(core_axis_name="c", subcore_axis_name="s")

    @functools.partial(
        pl.kernel,
        out_type=jax.ShapeDtypeStruct((n_tok, emb), jnp.float32),
        mesh=mesh,
        scratch_types=[
            pltpu.VMEM((n_per_w,), jnp.int32),
            pltpu.VMEM((chunk, emb), jnp.float32),
            pltpu.SemaphoreType.DMA,
        ],
    )
    def gather_kernel(tab_hbm, idx_hbm, o_hbm, idx_v, rows_v, sem):
        wid = lax.axis_index("s") * nc + lax.axis_index("c")
        base = wid * n_per_w
        pltpu.sync_copy(idx_hbm.at[pl.ds(base, n_per_w)], idx_v)

        @pl.loop(0, n_chunks)
        def _(j):
            off = j * chunk
            pltpu.async_copy(
                tab_hbm.at[idx_v.at[pl.ds(off, chunk)]], rows_v, sem
            ).wait()
            pltpu.sync_copy(rows_v, o_hbm.at[pl.ds(base + off, chunk)])

    return gather_kernel(word_emb, flat_ids)


def _tc_add_ln(gathered, tti_f, pe0, dt, gamma, beta, nb=32):
    b, s = tti_f.shape[:2]
    e = gathered.shape[-1]
    g3 = gathered.reshape(b, s, e)

    def body(g_ref, tti_ref, pe_ref, dt_ref, gam_ref, bet_ref, o_ref):
        x = g_ref[...] + pe_ref[...] + tti_ref[...] * dt_ref[...]
        mu = jnp.mean(x, axis=-1, keepdims=True)
        xc = x - mu
        var = jnp.mean(xc * xc, axis=-1, keepdims=True)
        y = xc * lax.rsqrt(var + _EPS)
        o_ref[...] = y * gam_ref[...] + bet_ref[...]

    return pl.pallas_call(
        body,
        grid=(b // nb,),
        in_specs=[
            pl.BlockSpec((nb, s, e), lambda i: (i, 0, 0)),
            pl.BlockSpec((nb, s, 1), lambda i: (i, 0, 0)),
            pl.BlockSpec((s, e), lambda i: (0, 0)),
            pl.BlockSpec((e,), lambda i: (0,)),
            pl.BlockSpec((e,), lambda i: (0,)),
            pl.BlockSpec((e,), lambda i: (0,)),
        ],
        out_specs=pl.BlockSpec((nb, s, e), lambda i: (i, 0, 0)),
        out_shape=jax.ShapeDtypeStruct((b, s, e), jnp.float32),
    )(g3, tti_f, pe0, dt, gamma, beta)


def kernel(token_ids, token_type_ids, word_emb, pos_emb, type_emb, ln_gamma, ln_beta):
    b, s = token_ids.shape
    e = word_emb.shape[1]
    flat_ids = token_ids.reshape(b * s).astype(jnp.int32)
    gathered = _sc_gather(word_emb, flat_ids)
    pe0 = lax.slice(pos_emb, (0, 0), (s, e)) + type_emb[0]
    dt = type_emb[1] - type_emb[0]
    tti_f = token_type_ids[..., None].astype(jnp.float32)
    return _tc_add_ln(gathered, tti_f, pe0, dt, ln_gamma, ln_beta)


# speedup vs baseline: 1.1087x; 1.0085x over previous
import functools

import jax
import jax.numpy as jnp
from jax import lax
from jax.experimental import pallas as pl
from jax.experimental.pallas import tpu as pltpu
from jax.experimental.pallas import tpu_sc as plsc

_EPS = 1e-12
_GATHER_WINDOW = 160


def _sc_gather(word_emb, flat_ids):
    n_tok = flat_ids.shape[0]
    emb = word_emb.shape[1]
    nc, ns = 2, 16
    nw = nc * ns
    n_per_w = n_tok // nw
    chunk = _GATHER_WINDOW
    while n_per_w % chunk:
        chunk -= 8
    n_chunks = n_per_w // chunk
    mesh = plsc.VectorSubcoreMesh(core_axis_name="c", subcore_axis_name="s")

    @functools.partial(
        pl.kernel,
        out_type=jax.ShapeDtypeStruct((n_tok, emb), jnp.float32),
        mesh=mesh,
        scratch_types=[
            pltpu.VMEM((n_per_w,), jnp.int32),
            pltpu.VMEM((chunk, emb), jnp.float32),
            pltpu.SemaphoreType.DMA,
        ],
    )
    def gather_kernel(tab_hbm, idx_hbm, o_hbm, idx_v, rows_v, sem):
        wid = lax.axis_index("s") * nc + lax.axis_index("c")
        base = wid * n_per_w
        pltpu.sync_copy(idx_hbm.at[pl.ds(base, n_per_w)], idx_v)

        @pl.loop(0, n_chunks)
        def _(j):
            off = j * chunk
            pltpu.async_copy(
                tab_hbm.at[idx_v.at[pl.ds(off, chunk)]], rows_v, sem
            ).wait()
            pltpu.sync_copy(rows_v, o_hbm.at[pl.ds(base + off, chunk)])

    return gather_kernel(word_emb, flat_ids)


def _tc_add_ln_slice(g3, tti_f, pe0, dt, gamma, beta, full_b, blk_off, prev=None,
                     nb=32):
    bs, s, e = g3.shape
    n_blk = bs // nb

    def body(g_ref, tti_ref, pe_ref, dt_ref, gam_ref, bet_ref, *rest):
        o_ref = rest[-1]
        x = g_ref[...] + pe_ref[...] + tti_ref[...] * dt_ref[...]
        mu = jnp.mean(x, axis=-1, keepdims=True)
        xc = x - mu
        var = jnp.mean(xc * xc, axis=-1, keepdims=True)
        y = xc * lax.rsqrt(var + _EPS)
        o_ref[...] = y * gam_ref[...] + bet_ref[...]

    in_specs = [
        pl.BlockSpec((nb, s, e), lambda i: (i, 0, 0)),
        pl.BlockSpec((nb, s, 1), lambda i: (i, 0, 0)),
        pl.BlockSpec((s, e), lambda i: (0, 0)),
        pl.BlockSpec((e,), lambda i: (0,)),
        pl.BlockSpec((e,), lambda i: (0,)),
        pl.BlockSpec((e,), lambda i: (0,)),
    ]
    args = [g3, tti_f, pe0, dt, gamma, beta]
    alias = {}
    if prev is not None:
        in_specs.append(pl.BlockSpec(memory_space=pl.ANY))
        args.append(prev)
        alias = {6: 0}
    return pl.pallas_call(
        body,
        grid=(n_blk,),
        in_specs=in_specs,
        out_specs=pl.BlockSpec((nb, s, e), lambda i: (blk_off + i, 0, 0)),
        out_shape=jax.ShapeDtypeStruct((full_b, s, e), jnp.float32),
        input_output_aliases=alias,
    )(*args)


_NSLICE = 4
_TC_NB = 32


def kernel(token_ids, token_type_ids, word_emb, pos_emb, type_emb, ln_gamma, ln_beta):
    b, s = token_ids.shape
    e = word_emb.shape[1]
    flat_ids = token_ids.reshape(b * s).astype(jnp.int32)
    pe0 = lax.slice(pos_emb, (0, 0), (s, e)) + type_emb[0]
    dt = type_emb[1] - type_emb[0]
    tti_f = token_type_ids[..., None].astype(jnp.float32)

    bs = b // _NSLICE
    out = None
    for i in range(_NSLICE):
        ids_i = lax.slice(flat_ids, (i * bs * s,), ((i + 1) * bs * s,))
        g_i = _sc_gather(word_emb, ids_i).reshape(bs, s, e)
        tti_i = lax.slice(tti_f, (i * bs, 0, 0), ((i + 1) * bs, s, 1))
        out = _tc_add_ln_slice(
            g_i, tti_i, pe0, dt, ln_gamma, ln_beta,
            full_b=b, blk_off=i * (bs // _TC_NB), prev=out, nb=_TC_NB,
        )
    return out
